# baseline (speedup 1.0000x reference)
import functools

import jax
import jax.numpy as jnp
from jax import lax
from jax.experimental import pallas as pl
from jax.experimental.pallas import tpu as pltpu
from jax.experimental.pallas import tpu_sc as plsc

VOCAB = 100000
D = 1024
PROMPT_LEN = 100
ID_OFFSET = 100000
B, S = 4, 2048
N = B * S
NC, NS, L = 2, 16, 16
NW = NC * NS
TPW = N // NW
CH = 32
NCH = TPW // CH

_mesh = plsc.VectorSubcoreMesh(core_axis_name="c", subcore_axis_name="s")


@functools.partial(
    pl.kernel,
    out_type=jax.ShapeDtypeStruct((B, S, D), jnp.float32),
    mesh=_mesh,
    scratch_types=[
        pltpu.VMEM((TPW,), jnp.int32),
        pltpu.VMEM((2, CH), jnp.int32),
        pltpu.VMEM((CH, D), jnp.float32),
        pltpu.VMEM((CH, D), jnp.float32),
        pltpu.SemaphoreType.DMA,
        pltpu.SemaphoreType.DMA,
        pltpu.SemaphoreType.DMA,
        pltpu.SemaphoreType.DMA,
    ],
)
def _embed_lookup(ids_hbm, w_hbm, p_hbm, out_hbm, ids_v, idx2_v,
                  rw0, rw1, semg0, semg1, semo0, semo1):
    wid = lax.axis_index("s") * NC + lax.axis_index("c")
    wb = wid // (S // TPW)
    ws = (wid % (S // TPW)) * TPW
    pltpu.sync_copy(ids_hbm.at[wb, pl.ds(ws, TPW)], ids_v)

    zeros = jnp.zeros((L,), jnp.int32)
    voff = jnp.full((L,), ID_OFFSET, jnp.int32)
    bufs = (rw0, rw1)
    semg = (semg0, semg1)
    semo = (semo0, semo1)

    def compute_idx(k, b):
        off = k * CH
        for j in range(CH // L):
            ids = ids_v[pl.ds(off + j * L, L)]
            pr = ids >= voff
            idx2_v[b, pl.ds(j * L, L)] = jnp.where(pr, zeros, ids)

    def patch_prompt_rows(k, rw):
        off = k * CH
        for j in range(CH // L):
            grp = ids_v[pl.ds(off + j * L, L)]
            for i in range(L):
                sid = grp[i]
                row = j * L + i

                @pl.when(sid >= ID_OFFSET)
                def _patch():
                    pltpu.sync_copy(
                        p_hbm.at[pl.ds(sid - ID_OFFSET, 1)],
                        rw.at[pl.ds(row, 1)])

    compute_idx(0, 0)
    gathers = [pltpu.async_copy(w_hbm.at[idx2_v.at[0]], rw0, semg0)]
    writes = []
    for k in range(NCH):
        b = k % 2
        nb = (k + 1) % 2
        if k + 1 < NCH:
            if k >= 1:
                writes[k - 1].wait()
            compute_idx(k + 1, nb)
            gathers.append(
                pltpu.async_copy(w_hbm.at[idx2_v.at[nb]], bufs[nb],
                                 semg[nb]))
        gathers[k].wait()
        patch_prompt_rows(k, bufs[b])
        writes.append(
            pltpu.async_copy(bufs[b],
                             out_hbm.at[wb, pl.ds(ws + k * CH, CH)], semo[b]))
    writes[NCH - 2].wait()
    writes[NCH - 1].wait()


def kernel(input_ids, labels, W, P):
    del labels
    return _embed_lookup(input_ids, W, P)


# speedup vs baseline: 1.0145x; 1.0145x over previous
import functools

import jax
import jax.numpy as jnp
from jax import lax
from jax.experimental import pallas as pl
from jax.experimental.pallas import tpu as pltpu
from jax.experimental.pallas import tpu_sc as plsc

VOCAB = 100000
D = 1024
PROMPT_LEN = 100
ID_OFFSET = 100000
B, S = 4, 2048
N = B * S
NC, NS, L = 2, 16, 16
NW = NC * NS
TPW = N // NW
CH = 32
NCH = TPW // CH

_mesh = plsc.VectorSubcoreMesh(core_axis_name="c", subcore_axis_name="s")


@functools.partial(
    pl.kernel,
    out_type=jax.ShapeDtypeStruct((B, S, D), jnp.float32),
    mesh=_mesh,
    scratch_types=[
        pltpu.VMEM((TPW,), jnp.int32),
        pltpu.VMEM((3, CH), jnp.int32),
        pltpu.VMEM((CH, D), jnp.float32),
        pltpu.VMEM((CH, D), jnp.float32),
        pltpu.VMEM((CH, D), jnp.float32),
        pltpu.SemaphoreType.DMA,
        pltpu.SemaphoreType.DMA,
        pltpu.SemaphoreType.DMA,
        pltpu.SemaphoreType.DMA,
        pltpu.SemaphoreType.DMA,
        pltpu.SemaphoreType.DMA,
    ],
)
def _embed_lookup(ids_hbm, w_hbm, p_hbm, out_hbm, ids_v, idx2_v,
                  rw0, rw1, rw2, semg0, semg1, semg2, semo0, semo1, semo2):
    wid = lax.axis_index("s") * NC + lax.axis_index("c")
    wb = wid // (S // TPW)
    ws = (wid % (S // TPW)) * TPW
    pltpu.sync_copy(ids_hbm.at[wb, pl.ds(ws, TPW)], ids_v)

    zeros = jnp.zeros((L,), jnp.int32)
    voff = jnp.full((L,), ID_OFFSET, jnp.int32)
    bufs = (rw0, rw1, rw2)
    semg = (semg0, semg1, semg2)
    semo = (semo0, semo1, semo2)
    NB = 3

    def compute_idx(k, b):
        off = k * CH
        for j in range(CH // L):
            ids = ids_v[pl.ds(off + j * L, L)]
            pr = ids >= voff
            idx2_v[b, pl.ds(j * L, L)] = jnp.where(pr, zeros, ids)

    def patch_prompt_rows(k, rw):
        off = k * CH
        for j in range(CH // L):
            grp = ids_v[pl.ds(off + j * L, L)]
            for i in range(L):
                sid = grp[i]
                row = j * L + i

                @pl.when(sid >= ID_OFFSET)
                def _patch():
                    pltpu.sync_copy(
                        p_hbm.at[pl.ds(sid - ID_OFFSET, 1)],
                        rw.at[pl.ds(row, 1)])

    compute_idx(0, 0)
    compute_idx(1, 1)
    gathers = [pltpu.async_copy(w_hbm.at[idx2_v.at[0]], rw0, semg0),
               pltpu.async_copy(w_hbm.at[idx2_v.at[1]], rw1, semg1)]
    writes = []
    for k in range(NCH):
        b = k % NB
        nb = (k + 2) % NB
        if k + 2 < NCH:
            if k >= 1:
                writes[k - 1].wait()
            compute_idx(k + 2, nb)
            gathers.append(
                pltpu.async_copy(w_hbm.at[idx2_v.at[nb]], bufs[nb],
                                 semg[nb]))
        gathers[k].wait()
        patch_prompt_rows(k, bufs[b])
        writes.append(
            pltpu.async_copy(bufs[b],
                             out_hbm.at[wb, pl.ds(ws + k * CH, CH)], semo[b]))
    writes[NCH - 3].wait()
    writes[NCH - 2].wait()
    writes[NCH - 1].wait()


def kernel(input_ids, labels, W, P):
    del labels
    return _embed_lookup(input_ids, W, P)


# speedup vs baseline: 1.1808x; 1.1639x over previous
import functools

import jax
import jax.numpy as jnp
from jax import lax
from jax.experimental import pallas as pl
from jax.experimental.pallas import tpu as pltpu
from jax.experimental.pallas import tpu_sc as plsc

VOCAB = 100000
D = 1024
PROMPT_LEN = 100
ID_OFFSET = 100000
B, S = 4, 2048
N = B * S
NC, NS, L = 2, 16, 16
NW = NC * NS
TPW = N // NW
CH = 16
NCH = TPW // CH
NB = 4

_mesh = plsc.VectorSubcoreMesh(core_axis_name="c", subcore_axis_name="s")


@functools.partial(
    pl.kernel,
    out_type=jax.ShapeDtypeStruct((B, S, D), jnp.float32),
    mesh=_mesh,
    scratch_types=[
        pltpu.VMEM((TPW,), jnp.int32),
        pltpu.VMEM((NB, CH), jnp.int32),
        pltpu.VMEM((CH, D), jnp.float32),
        pltpu.VMEM((CH, D), jnp.float32),
        pltpu.VMEM((CH, D), jnp.float32),
        pltpu.VMEM((CH, D), jnp.float32),
        pltpu.SemaphoreType.DMA,
        pltpu.SemaphoreType.DMA,
        pltpu.SemaphoreType.DMA,
        pltpu.SemaphoreType.DMA,
        pltpu.SemaphoreType.DMA,
        pltpu.SemaphoreType.DMA,
        pltpu.SemaphoreType.DMA,
        pltpu.SemaphoreType.DMA,
    ],
)
def _embed_lookup(ids_hbm, w_hbm, p_hbm, out_hbm, ids_v, idx_v,
                  rw0, rw1, rw2, rw3,
                  semg0, semg1, semg2, semg3, semo0, semo1, semo2, semo3):
    wid = lax.axis_index("s") * NC + lax.axis_index("c")
    wb = wid // (S // TPW)
    ws = (wid % (S // TPW)) * TPW
    pltpu.sync_copy(ids_hbm.at[wb, pl.ds(ws, TPW)], ids_v)

    zeros = jnp.zeros((L,), jnp.int32)
    voff = jnp.full((L,), ID_OFFSET, jnp.int32)
    bufs = (rw0, rw1, rw2, rw3)
    semg = (semg0, semg1, semg2, semg3)
    semo = (semo0, semo1, semo2, semo3)

    def compute_idx(off, b):
        ids = ids_v[pl.ds(off, L)]
        pr = ids >= voff
        idx_v[b, pl.ds(0, L)] = jnp.where(pr, zeros, ids)

    def issue_gather(b):
        return pltpu.async_copy(w_hbm.at[idx_v.at[b]], bufs[b], semg[b])

    def issue_write(off, b):
        return pltpu.async_copy(bufs[b], out_hbm.at[wb, pl.ds(ws + off, CH)],
                                semo[b])

    def patch_prompt_rows(off, rw):
        grp = ids_v[pl.ds(off, L)]
        for i in range(L):
            sid = grp[i]

            @pl.when(sid >= ID_OFFSET)
            def _patch():
                pltpu.sync_copy(
                    p_hbm.at[pl.ds(sid - ID_OFFSET, 1)],
                    rw.at[pl.ds(i, 1)])

    for c in range(NB - 1):
        compute_idx(c * CH, c)
        issue_gather(c)

    def ring_step(g, carry):
        for b in range(NB):
            c = g * NB + b
            off = c * CH

            @pl.when(c + NB - 1 < NCH)
            def _prefetch():
                pb = (b + NB - 1) % NB
                poff = (c + NB - 1) * CH

                @pl.when(c >= 1)
                def _drain_prev_write():
                    pltpu.make_async_copy(
                        bufs[pb],
                        out_hbm.at[wb, pl.ds(ws + (c - 1) * CH, CH)],
                        semo[pb]).wait()

                compute_idx(poff, pb)
                issue_gather(pb)

            pltpu.make_async_copy(w_hbm.at[idx_v.at[b]], bufs[b],
                                  semg[b]).wait()
            patch_prompt_rows(off, bufs[b])
            issue_write(off, b)
        return carry

    lax.fori_loop(0, NCH // NB, ring_step, 0)

    for c in range(NCH - NB, NCH):
        pltpu.make_async_copy(
            bufs[c % NB],
            out_hbm.at[wb, pl.ds(ws + c * CH, CH)],
            semo[c % NB]).wait()


def kernel(input_ids, labels, W, P):
    del labels
    return _embed_lookup(input_ids, W, P)


# speedup vs baseline: 1.1940x; 1.0112x over previous
import functools

import jax
import jax.numpy as jnp
from jax import lax
from jax.experimental import pallas as pl
from jax.experimental.pallas import tpu as pltpu
from jax.experimental.pallas import tpu_sc as plsc

VOCAB = 100000
D = 1024
PROMPT_LEN = 100
ID_OFFSET = 100000
B, S = 4, 2048
N = B * S
NC, NS, L = 2, 16, 16
NW = NC * NS
TPW = N // NW
CH = 16
NCH = TPW // CH
NB = 4

_mesh = plsc.VectorSubcoreMesh(core_axis_name="c", subcore_axis_name="s")


@functools.partial(
    pl.kernel,
    out_type=jax.ShapeDtypeStruct((B, S, D), jnp.float32),
    mesh=_mesh,
    scratch_types=[
        pltpu.VMEM((TPW,), jnp.int32),
        pltpu.VMEM((NB, CH), jnp.int32),
        pltpu.VMEM((CH, D), jnp.float32),
        pltpu.VMEM((CH, D), jnp.float32),
        pltpu.VMEM((CH, D), jnp.float32),
        pltpu.VMEM((CH, D), jnp.float32),
        pltpu.SemaphoreType.DMA,
        pltpu.SemaphoreType.DMA,
        pltpu.SemaphoreType.DMA,
        pltpu.SemaphoreType.DMA,
        pltpu.SemaphoreType.DMA,
        pltpu.SemaphoreType.DMA,
        pltpu.SemaphoreType.DMA,
        pltpu.SemaphoreType.DMA,
    ],
)
def _embed_lookup(ids_hbm, w_hbm, p_hbm, out_hbm, ids_v, idx_v,
                  rw0, rw1, rw2, rw3,
                  semg0, semg1, semg2, semg3, semo0, semo1, semo2, semo3):
    wid = lax.axis_index("s") * NC + lax.axis_index("c")
    wb = wid // (S // TPW)
    ws = (wid % (S // TPW)) * TPW
    pltpu.sync_copy(ids_hbm.at[wb, pl.ds(ws, TPW)], ids_v)

    zeros = jnp.zeros((L,), jnp.int32)
    voff = jnp.full((L,), ID_OFFSET, jnp.int32)
    bufs = (rw0, rw1, rw2, rw3)
    semg = (semg0, semg1, semg2, semg3)
    semo = (semo0, semo1, semo2, semo3)

    def compute_idx(off, b):
        ids = ids_v[pl.ds(off, L)]
        pr = ids >= voff
        idx_v[b, pl.ds(0, L)] = jnp.where(pr, zeros, ids)

    def issue_gather(b):
        return pltpu.async_copy(w_hbm.at[idx_v.at[b]], bufs[b], semg[b])

    def issue_write(off, b):
        return pltpu.async_copy(bufs[b], out_hbm.at[wb, pl.ds(ws + off, CH)],
                                semo[b])

    def patch_prompt_rows(off, rw):
        grp = ids_v[pl.ds(off, L)]
        for i in range(L):
            sid = grp[i]

            @pl.when(sid >= ID_OFFSET)
            def _patch():
                pltpu.sync_copy(
                    p_hbm.at[pl.ds(sid - ID_OFFSET, 1)],
                    rw.at[pl.ds(i, 1)])

    K = 2
    for c in range(K):
        compute_idx(c * CH, c)
        issue_gather(c)

    def ring_step(g, carry):
        for b in range(NB):
            c = g * NB + b
            off = c * CH

            @pl.when(c + K < NCH)
            def _prefetch():
                pb = (b + K) % NB
                poff = (c + K) * CH

                @pl.when(c >= NB - K)
                def _drain_prev_write():
                    pltpu.make_async_copy(
                        bufs[pb],
                        out_hbm.at[wb, pl.ds(ws + (c + K - NB) * CH, CH)],
                        semo[pb]).wait()

                compute_idx(poff, pb)
                issue_gather(pb)

            pltpu.make_async_copy(w_hbm.at[idx_v.at[b]], bufs[b],
                                  semg[b]).wait()
            patch_prompt_rows(off, bufs[b])
            issue_write(off, b)
        return carry

    lax.fori_loop(0, NCH // NB, ring_step, 0)

    for c in range(NCH - NB, NCH):
        pltpu.make_async_copy(
            bufs[c % NB],
            out_hbm.at[wb, pl.ds(ws + c * CH, CH)],
            semo[c % NB]).wait()


def kernel(input_ids, labels, W, P):
    del labels
    return _embed_lookup(input_ids, W, P)
